# Initial kernel scaffold; baseline (speedup 1.0000x reference)
import jax
import jax.numpy as jnp
from jax.experimental import pallas as pl


def kernel(features, W_proj, gamma, beta, memory_bank):
    raise NotImplementedError("write your pallas kernel here")


# speedup vs baseline: 34.8499x; 34.8499x over previous
import jax
import jax.numpy as jnp
from jax.experimental import pallas as pl
from jax.experimental.pallas import tpu as pltpu

FEATURE_DIM = 256
MEMORY_SIZE = 8192
K_NEIGHBORS = 20
NUM_CLUSTERS = 8
TEMPERATURE = 0.1
BN_EPS = 1e-5

_B = 4
_N = 1024
_NBLK = 256

_HI = jax.lax.Precision.HIGHEST


def _project_kernel(f_ref, w_ref, g_ref, b_ref, x_ref):
    w = w_ref[...]
    for b in range(_B):
        y = jnp.dot(w, f_ref[b], preferred_element_type=jnp.float32,
                    precision=_HI)
        x_ref[b] = y
    s1 = jnp.zeros((FEATURE_DIM, 1), jnp.float32)
    for b in range(_B):
        s1 = s1 + jnp.sum(x_ref[b], axis=1, keepdims=True)
    mean = s1 / float(_B * _N)
    s2 = jnp.zeros((FEATURE_DIM, 1), jnp.float32)
    for b in range(_B):
        d = x_ref[b] - mean
        s2 = s2 + jnp.sum(d * d, axis=1, keepdims=True)
    var = s2 / float(_B * _N)
    scale = g_ref[...] / jnp.sqrt(var + BN_EPS)
    shift = b_ref[...] - mean * scale
    for b in range(_B):
        x_ref[b] = jnp.maximum(x_ref[b] * scale + shift, 0.0)


_MCHUNK = 1024
_NMC = MEMORY_SIZE // _MCHUNK


def _density_kernel(x3_ref, mem_ref, dens_ref, d2_ref):
    xb = x3_ref[...]
    x2 = jnp.sum(xb * xb, axis=0, keepdims=True)
    for j in range(_NMC):
        mem_c = mem_ref[pl.ds(j * _MCHUNK, _MCHUNK), :]
        m2 = jnp.sum(mem_c * mem_c, axis=1, keepdims=True)
        mm = jnp.dot(mem_c, xb, preferred_element_type=jnp.float32,
                     precision=_HI)
        d2_ref[pl.ds(j * _MCHUNK, _MCHUNK), :] = (
            jnp.maximum(m2 + x2 - 2.0 * mm, 1e-12))
    big = jnp.float32(3.0e38)
    t = jnp.zeros((1, _NBLK), jnp.float32)
    rk2 = jnp.zeros((1, _NBLK), jnp.float32)
    found = jnp.zeros((1, _NBLK), jnp.bool_)
    for _ in range(K_NEIGHBORS):
        nxt = jnp.full((1, _NBLK), big, jnp.float32)
        cnt = jnp.zeros((1, _NBLK), jnp.float32)
        for j in range(_NMC):
            c = d2_ref[pl.ds(j * _MCHUNK, _MCHUNK), :]
            le = c <= t
            nxt = jnp.minimum(
                nxt, jnp.min(jnp.where(le, big, c), axis=0, keepdims=True))
            cnt = cnt + jnp.sum(le.astype(jnp.float32), axis=0, keepdims=True)
        newly = jnp.logical_and(jnp.logical_not(found),
                                cnt >= float(K_NEIGHBORS))
        rk2 = jnp.where(newly, t, rk2)
        found = jnp.logical_or(found, newly)
        t = nxt
    cnt = jnp.zeros((1, _NBLK), jnp.float32)
    for j in range(_NMC):
        c = d2_ref[pl.ds(j * _MCHUNK, _MCHUNK), :]
        cnt = cnt + jnp.sum((c <= t).astype(jnp.float32), axis=0,
                            keepdims=True)
    newly = jnp.logical_and(jnp.logical_not(found), cnt >= float(K_NEIGHBORS))
    rk2 = jnp.where(newly, t, rk2)
    bw2 = jnp.maximum(rk2, 1e-12)
    acc = jnp.zeros((1, _NBLK), jnp.float32)
    for j in range(_NMC):
        c = d2_ref[pl.ds(j * _MCHUNK, _MCHUNK), :]
        acc = acc + jnp.sum(jnp.exp(-c / bw2), axis=0, keepdims=True)
    dens_ref[...] = acc


def _assign_kernel(x_ref, dens_ref, sem_ref):
    dens = dens_ref[...]
    iota = jax.lax.broadcasted_iota(jnp.int32, (1, _N), 1)
    onehots = []
    vals = []
    for _ in range(NUM_CLUSTERS):
        mx = jnp.max(dens)
        idx = jnp.min(jnp.where(dens == mx, iota, jnp.int32(2 ** 30)))
        oh = (iota == idx)
        onehots.append(oh.astype(jnp.float32))
        vals.append(mx)
        dens = jnp.where(oh, jnp.float32(-3.0e38), dens)
    onehot = jnp.concatenate(onehots, axis=0)
    cdens = jnp.stack(vals).reshape(NUM_CLUSTERS, 1)
    centers = jax.lax.dot_general(
        onehot, x_ref[_B - 1], (((1,), (1,)), ((), ())),
        preferred_element_type=jnp.float32, precision=_HI)
    priors = cdens / (jnp.sum(cdens) + 1e-8)
    c2 = jnp.sum(centers * centers, axis=1, keepdims=True)
    rows = []
    for b in range(_B):
        xb = x_ref[b]
        x2 = jnp.sum(xb * xb, axis=0, keepdims=True)
        cm = jnp.dot(centers, xb, preferred_element_type=jnp.float32,
                     precision=_HI)
        d2c = jnp.maximum(c2 + x2 - 2.0 * cm, 1e-12)
        logits = -jnp.sqrt(d2c) / TEMPERATURE
        mxl = jnp.max(logits, axis=0, keepdims=True)
        e = jnp.exp(logits - mxl)
        s = jnp.sum(e, axis=0, keepdims=True)
        rows.append(jnp.sum(priors * e, axis=0, keepdims=True) / s)
    sem_ref[...] = jnp.concatenate(rows, axis=0)


def kernel(features, W_proj, gamma, beta, memory_bank):
    B, C, H, W = features.shape
    f = features.reshape(B, C, H * W)
    g = gamma.reshape(C, 1)
    bt = beta.reshape(C, 1)

    x = pl.pallas_call(
        _project_kernel,
        out_shape=jax.ShapeDtypeStruct((B, C, H * W), jnp.float32),
    )(f, W_proj, g, bt)

    nblocks = (H * W) // _NBLK
    density = pl.pallas_call(
        _density_kernel,
        grid=(nblocks,),
        in_specs=[
            pl.BlockSpec((C, _NBLK), lambda i: (0, i)),
            pl.BlockSpec((MEMORY_SIZE, C), lambda i: (0, 0)),
        ],
        out_specs=pl.BlockSpec((1, _NBLK), lambda i: (0, i)),
        out_shape=jax.ShapeDtypeStruct((1, H * W), jnp.float32),
        scratch_shapes=[pltpu.VMEM((MEMORY_SIZE, _NBLK), jnp.float32)],
    )(x[B - 1], memory_bank)

    sem = pl.pallas_call(
        _assign_kernel,
        out_shape=jax.ShapeDtypeStruct((B, H * W), jnp.float32),
    )(x, density)

    return sem.reshape(B, 1, H, W)


# speedup vs baseline: 41.1426x; 1.1806x over previous
import jax
import jax.numpy as jnp
from jax.experimental import pallas as pl
from jax.experimental.pallas import tpu as pltpu

FEATURE_DIM = 256
MEMORY_SIZE = 8192
K_NEIGHBORS = 20
NUM_CLUSTERS = 8
TEMPERATURE = 0.1
BN_EPS = 1e-5

_B = 4
_N = 1024
_NBLK = 512

_HI = jax.lax.Precision.HIGHEST


def _project_kernel(f_ref, w_ref, g_ref, b_ref, x_ref):
    w = w_ref[...]
    for b in range(_B):
        y = jnp.dot(w, f_ref[b], preferred_element_type=jnp.float32,
                    precision=_HI)
        x_ref[b] = y
    s1 = jnp.zeros((FEATURE_DIM, 1), jnp.float32)
    for b in range(_B):
        s1 = s1 + jnp.sum(x_ref[b], axis=1, keepdims=True)
    mean = s1 / float(_B * _N)
    s2 = jnp.zeros((FEATURE_DIM, 1), jnp.float32)
    for b in range(_B):
        d = x_ref[b] - mean
        s2 = s2 + jnp.sum(d * d, axis=1, keepdims=True)
    var = s2 / float(_B * _N)
    scale = g_ref[...] / jnp.sqrt(var + BN_EPS)
    shift = b_ref[...] - mean * scale
    for b in range(_B):
        x_ref[b] = jnp.maximum(x_ref[b] * scale + shift, 0.0)


_MCHUNK = 1024
_NMC = MEMORY_SIZE // _MCHUNK


def _density_kernel(x3_ref, mem_ref, dens_ref, d2_ref):
    xb = x3_ref[...]
    x2 = jnp.sum(xb * xb, axis=0, keepdims=True)
    for j in range(_NMC):
        mem_c = mem_ref[pl.ds(j * _MCHUNK, _MCHUNK), :]
        m2 = jnp.sum(mem_c * mem_c, axis=1, keepdims=True)
        mm = jnp.dot(mem_c, xb,
                     preferred_element_type=jnp.float32)
        d2_ref[pl.ds(j * _MCHUNK, _MCHUNK), :] = (
            jnp.maximum(m2 + x2 - 2.0 * mm, 1e-12))
    big = jnp.float32(3.0e38)
    t = jnp.zeros((1, _NBLK), jnp.float32)
    rk2 = jnp.zeros((1, _NBLK), jnp.float32)
    found = jnp.zeros((1, _NBLK), jnp.bool_)
    for _ in range(K_NEIGHBORS):
        nxt = jnp.full((1, _NBLK), big, jnp.float32)
        cnt = jnp.zeros((1, _NBLK), jnp.float32)
        for j in range(_NMC):
            c = d2_ref[pl.ds(j * _MCHUNK, _MCHUNK), :]
            le = c <= t
            nxt = jnp.minimum(
                nxt, jnp.min(jnp.where(le, big, c), axis=0, keepdims=True))
            cnt = cnt + jnp.sum(le.astype(jnp.float32), axis=0, keepdims=True)
        newly = jnp.logical_and(jnp.logical_not(found),
                                cnt >= float(K_NEIGHBORS))
        rk2 = jnp.where(newly, t, rk2)
        found = jnp.logical_or(found, newly)
        t = nxt
    rk2 = jnp.where(found, rk2, t)
    neg_inv_bw2 = -1.0 / jnp.maximum(rk2, 1e-12)
    acc = jnp.zeros((1, _NBLK), jnp.float32)
    for j in range(_NMC):
        c = d2_ref[pl.ds(j * _MCHUNK, _MCHUNK), :]
        acc = acc + jnp.sum(jnp.exp(c * neg_inv_bw2), axis=0, keepdims=True)
    dens_ref[...] = acc


def _assign_kernel(x_ref, dens_ref, sem_ref):
    dens = dens_ref[...]
    iota = jax.lax.broadcasted_iota(jnp.int32, (1, _N), 1)
    onehots = []
    vals = []
    for _ in range(NUM_CLUSTERS):
        mx = jnp.max(dens)
        idx = jnp.min(jnp.where(dens == mx, iota, jnp.int32(2 ** 30)))
        oh = (iota == idx)
        onehots.append(oh.astype(jnp.float32))
        vals.append(mx)
        dens = jnp.where(oh, jnp.float32(-3.0e38), dens)
    onehot = jnp.concatenate(onehots, axis=0)
    cdens = jnp.stack(vals).reshape(NUM_CLUSTERS, 1)
    centers = jax.lax.dot_general(
        onehot, x_ref[_B - 1], (((1,), (1,)), ((), ())),
        preferred_element_type=jnp.float32, precision=_HI)
    priors = cdens / (jnp.sum(cdens) + 1e-8)
    c2 = jnp.sum(centers * centers, axis=1, keepdims=True)
    rows = []
    for b in range(_B):
        xb = x_ref[b]
        x2 = jnp.sum(xb * xb, axis=0, keepdims=True)
        cm = jnp.dot(centers, xb, preferred_element_type=jnp.float32,
                     precision=_HI)
        d2c = jnp.maximum(c2 + x2 - 2.0 * cm, 1e-12)
        logits = -jnp.sqrt(d2c) / TEMPERATURE
        mxl = jnp.max(logits, axis=0, keepdims=True)
        e = jnp.exp(logits - mxl)
        s = jnp.sum(e, axis=0, keepdims=True)
        rows.append(jnp.sum(priors * e, axis=0, keepdims=True) / s)
    sem_ref[...] = jnp.concatenate(rows, axis=0)


def kernel(features, W_proj, gamma, beta, memory_bank):
    B, C, H, W = features.shape
    f = features.reshape(B, C, H * W)
    g = gamma.reshape(C, 1)
    bt = beta.reshape(C, 1)

    x = pl.pallas_call(
        _project_kernel,
        out_shape=jax.ShapeDtypeStruct((B, C, H * W), jnp.float32),
    )(f, W_proj, g, bt)

    nblocks = (H * W) // _NBLK
    density = pl.pallas_call(
        _density_kernel,
        grid=(nblocks,),
        in_specs=[
            pl.BlockSpec((C, _NBLK), lambda i: (0, i)),
            pl.BlockSpec((MEMORY_SIZE, C), lambda i: (0, 0)),
        ],
        out_specs=pl.BlockSpec((1, _NBLK), lambda i: (0, i)),
        out_shape=jax.ShapeDtypeStruct((1, H * W), jnp.float32),
        scratch_shapes=[pltpu.VMEM((MEMORY_SIZE, _NBLK), jnp.float32)],
    )(x[B - 1], memory_bank)

    sem = pl.pallas_call(
        _assign_kernel,
        out_shape=jax.ShapeDtypeStruct((B, H * W), jnp.float32),
    )(x, density)

    return sem.reshape(B, 1, H, W)


# speedup vs baseline: 41.8961x; 1.0183x over previous
import jax
import jax.numpy as jnp
from jax.experimental import pallas as pl
from jax.experimental.pallas import tpu as pltpu

FEATURE_DIM = 256
MEMORY_SIZE = 8192
K_NEIGHBORS = 20
NUM_CLUSTERS = 8
TEMPERATURE = 0.1
BN_EPS = 1e-5

_B = 4
_N = 1024
_NBLK = 512

_HI = jax.lax.Precision.HIGHEST


def _project_kernel(f_ref, w_ref, g_ref, b_ref, x_ref):
    w = w_ref[...]
    for b in range(_B):
        y = jnp.dot(w, f_ref[b], preferred_element_type=jnp.float32,
                    precision=_HI)
        x_ref[b] = y
    s1 = jnp.zeros((FEATURE_DIM, 1), jnp.float32)
    for b in range(_B):
        s1 = s1 + jnp.sum(x_ref[b], axis=1, keepdims=True)
    mean = s1 / float(_B * _N)
    s2 = jnp.zeros((FEATURE_DIM, 1), jnp.float32)
    for b in range(_B):
        d = x_ref[b] - mean
        s2 = s2 + jnp.sum(d * d, axis=1, keepdims=True)
    var = s2 / float(_B * _N)
    scale = g_ref[...] / jnp.sqrt(var + BN_EPS)
    shift = b_ref[...] - mean * scale
    for b in range(_B):
        x_ref[b] = jnp.maximum(x_ref[b] * scale + shift, 0.0)


_MCHUNK = 1024
_NMC = MEMORY_SIZE // _MCHUNK


def _density_kernel(x3_ref, mem_ref, dens_ref, d2_ref, st_ref):
    xb = x3_ref[...]
    x2 = jnp.sum(xb * xb, axis=0, keepdims=True)
    for j in range(_NMC):
        mem_c = mem_ref[pl.ds(j * _MCHUNK, _MCHUNK), :]
        m2 = jnp.sum(mem_c * mem_c, axis=1, keepdims=True)
        mm = jnp.dot(mem_c, xb,
                     preferred_element_type=jnp.float32)
        d2_ref[pl.ds(j * _MCHUNK, _MCHUNK), :] = (
            jnp.maximum(m2 + x2 - 2.0 * mm, 1e-12))
    big = jnp.float32(3.0e38)
    kf = float(K_NEIGHBORS)
    t = jnp.zeros((1, _NBLK), jnp.float32)
    tau = jnp.zeros((1, _NBLK), jnp.float32)
    found = jnp.zeros((1, _NBLK), jnp.bool_)
    for _ in range(K_NEIGHBORS):
        c0 = d2_ref[pl.ds(0, _MCHUNK), :]
        le = c0 <= t
        nxt = jnp.min(jnp.where(le, big, c0), axis=0, keepdims=True)
        cnt = jnp.sum(le.astype(jnp.float32), axis=0, keepdims=True)
        newly = jnp.logical_and(jnp.logical_not(found), cnt >= kf)
        tau = jnp.where(newly, t, tau)
        found = jnp.logical_or(found, newly)
        t = nxt
    tau = jnp.where(found, tau, t)
    lo = jnp.zeros((1, _NBLK), jnp.float32)
    hi = tau
    for _ in range(3):
        mid = 0.5 * (lo + hi)
        cnt = jnp.zeros((1, _NBLK), jnp.float32)
        for j in range(_NMC):
            c = d2_ref[pl.ds(j * _MCHUNK, _MCHUNK), :]
            cnt = cnt + jnp.sum((c <= mid).astype(jnp.float32), axis=0,
                                keepdims=True)
        ge = cnt >= kf
        hi = jnp.where(ge, mid, hi)
        lo = jnp.where(ge, lo, mid)
    st_ref[0:1, :] = lo
    st_ref[1:2, :] = jnp.zeros((1, _NBLK), jnp.float32)
    st_ref[2:3, :] = jnp.zeros((1, _NBLK), jnp.float32)

    def scan_body(carry):
        i, _ = carry
        t = st_ref[0:1, :]
        rk2 = st_ref[1:2, :]
        fnd = st_ref[2:3, :]
        nxt = jnp.full((1, _NBLK), big, jnp.float32)
        cnt = jnp.zeros((1, _NBLK), jnp.float32)
        for j in range(_NMC):
            c = d2_ref[pl.ds(j * _MCHUNK, _MCHUNK), :]
            le = c <= t
            nxt = jnp.minimum(
                nxt, jnp.min(jnp.where(le, big, c), axis=0, keepdims=True))
            cnt = cnt + jnp.sum(le.astype(jnp.float32), axis=0, keepdims=True)
        newly = jnp.logical_and(fnd == 0.0, cnt >= kf)
        rk2 = jnp.where(newly, t, rk2)
        fnd = jnp.where(newly, 1.0, fnd)
        st_ref[0:1, :] = jnp.where(fnd > 0.0, t, nxt)
        st_ref[1:2, :] = rk2
        st_ref[2:3, :] = fnd
        return i + 1, jnp.all(fnd > 0.0)

    def scan_cond(carry):
        i, done = carry
        return jnp.logical_and(i < K_NEIGHBORS + 2, jnp.logical_not(done))

    jax.lax.while_loop(scan_cond, scan_body, (jnp.int32(0), False))
    rk2 = jnp.where(st_ref[2:3, :] > 0.0, st_ref[1:2, :], st_ref[0:1, :])
    neg_inv_bw2 = -1.0 / jnp.maximum(rk2, 1e-12)
    acc = jnp.zeros((1, _NBLK), jnp.float32)
    for j in range(_NMC):
        c = d2_ref[pl.ds(j * _MCHUNK, _MCHUNK), :]
        acc = acc + jnp.sum(jnp.exp(c * neg_inv_bw2), axis=0, keepdims=True)
    dens_ref[...] = acc


def _assign_kernel(x_ref, dens_ref, sem_ref):
    dens = dens_ref[...]
    iota = jax.lax.broadcasted_iota(jnp.int32, (1, _N), 1)
    onehots = []
    vals = []
    for _ in range(NUM_CLUSTERS):
        mx = jnp.max(dens)
        idx = jnp.min(jnp.where(dens == mx, iota, jnp.int32(2 ** 30)))
        oh = (iota == idx)
        onehots.append(oh.astype(jnp.float32))
        vals.append(mx)
        dens = jnp.where(oh, jnp.float32(-3.0e38), dens)
    onehot = jnp.concatenate(onehots, axis=0)
    cdens = jnp.stack(vals).reshape(NUM_CLUSTERS, 1)
    centers = jax.lax.dot_general(
        onehot, x_ref[_B - 1], (((1,), (1,)), ((), ())),
        preferred_element_type=jnp.float32, precision=_HI)
    priors = cdens / (jnp.sum(cdens) + 1e-8)
    c2 = jnp.sum(centers * centers, axis=1, keepdims=True)
    rows = []
    for b in range(_B):
        xb = x_ref[b]
        x2 = jnp.sum(xb * xb, axis=0, keepdims=True)
        cm = jnp.dot(centers, xb, preferred_element_type=jnp.float32,
                     precision=_HI)
        d2c = jnp.maximum(c2 + x2 - 2.0 * cm, 1e-12)
        logits = -jnp.sqrt(d2c) / TEMPERATURE
        mxl = jnp.max(logits, axis=0, keepdims=True)
        e = jnp.exp(logits - mxl)
        s = jnp.sum(e, axis=0, keepdims=True)
        rows.append(jnp.sum(priors * e, axis=0, keepdims=True) / s)
    sem_ref[...] = jnp.concatenate(rows, axis=0)


def kernel(features, W_proj, gamma, beta, memory_bank):
    B, C, H, W = features.shape
    f = features.reshape(B, C, H * W)
    g = gamma.reshape(C, 1)
    bt = beta.reshape(C, 1)

    x = pl.pallas_call(
        _project_kernel,
        out_shape=jax.ShapeDtypeStruct((B, C, H * W), jnp.float32),
    )(f, W_proj, g, bt)

    nblocks = (H * W) // _NBLK
    density = pl.pallas_call(
        _density_kernel,
        grid=(nblocks,),
        in_specs=[
            pl.BlockSpec((C, _NBLK), lambda i: (0, i)),
            pl.BlockSpec((MEMORY_SIZE, C), lambda i: (0, 0)),
        ],
        out_specs=pl.BlockSpec((1, _NBLK), lambda i: (0, i)),
        out_shape=jax.ShapeDtypeStruct((1, H * W), jnp.float32),
        scratch_shapes=[pltpu.VMEM((MEMORY_SIZE, _NBLK), jnp.float32),
                        pltpu.VMEM((8, _NBLK), jnp.float32)],
    )(x[B - 1], memory_bank)

    sem = pl.pallas_call(
        _assign_kernel,
        out_shape=jax.ShapeDtypeStruct((B, H * W), jnp.float32),
    )(x, density)

    return sem.reshape(B, 1, H, W)


# speedup vs baseline: 49.7771x; 1.1881x over previous
import jax
import jax.numpy as jnp
from jax.experimental import pallas as pl
from jax.experimental.pallas import tpu as pltpu

FEATURE_DIM = 256
MEMORY_SIZE = 8192
K_NEIGHBORS = 20
NUM_CLUSTERS = 8
TEMPERATURE = 0.1
BN_EPS = 1e-5

_B = 4
_N = 1024
_NBLK = 512

_HI = jax.lax.Precision.HIGHEST


def _project_kernel(f_ref, w_ref, g_ref, b_ref, x_ref):
    w = w_ref[...]
    for b in range(_B):
        y = jnp.dot(w, f_ref[b], preferred_element_type=jnp.float32,
                    precision=_HI)
        x_ref[b] = y
    s1 = jnp.zeros((FEATURE_DIM, 1), jnp.float32)
    for b in range(_B):
        s1 = s1 + jnp.sum(x_ref[b], axis=1, keepdims=True)
    mean = s1 / float(_B * _N)
    s2 = jnp.zeros((FEATURE_DIM, 1), jnp.float32)
    for b in range(_B):
        d = x_ref[b] - mean
        s2 = s2 + jnp.sum(d * d, axis=1, keepdims=True)
    var = s2 / float(_B * _N)
    scale = g_ref[...] / jnp.sqrt(var + BN_EPS)
    shift = b_ref[...] - mean * scale
    for b in range(_B):
        x_ref[b] = jnp.maximum(x_ref[b] * scale + shift, 0.0)


_MCHUNK = 1024
_NMC = MEMORY_SIZE // _MCHUNK


def _density_kernel(x3_ref, mem_ref, dens_ref, d2_ref, st_ref):
    xb = x3_ref[...]
    x2 = jnp.sum(xb * xb, axis=0, keepdims=True)
    for j in range(_NMC):
        mem_c = mem_ref[pl.ds(j * _MCHUNK, _MCHUNK), :]
        m2 = jnp.sum(mem_c * mem_c, axis=1, keepdims=True)
        mm = jnp.dot(mem_c, xb,
                     preferred_element_type=jnp.float32)
        d2_ref[pl.ds(j * _MCHUNK, _MCHUNK), :] = (
            jnp.maximum(m2 + x2 - 2.0 * mm, 1e-12))
    big = jnp.float32(3.0e38)
    kf = float(K_NEIGHBORS)
    t = jnp.zeros((1, _NBLK), jnp.float32)
    tau = jnp.zeros((1, _NBLK), jnp.float32)
    found = jnp.zeros((1, _NBLK), jnp.bool_)
    for _ in range(K_NEIGHBORS):
        c0 = d2_ref[pl.ds(0, _MCHUNK), :]
        le = c0 <= t
        nxt = jnp.min(jnp.where(le, big, c0), axis=0, keepdims=True)
        cnt = jnp.sum(le.astype(jnp.float32), axis=0, keepdims=True)
        newly = jnp.logical_and(jnp.logical_not(found), cnt >= kf)
        tau = jnp.where(newly, t, tau)
        found = jnp.logical_or(found, newly)
        t = nxt
    tau = jnp.where(found, tau, t)
    lo = jnp.zeros((1, _NBLK), jnp.float32)
    hi = tau
    for _ in range(8):
        mid = 0.5 * (lo + hi)
        cnt = jnp.zeros((1, _NBLK), jnp.float32)
        for j in range(_NMC):
            c = d2_ref[pl.ds(j * _MCHUNK, _MCHUNK), :]
            cnt = cnt + jnp.sum((c <= mid).astype(jnp.float32), axis=0,
                                keepdims=True)
        ge = cnt >= kf
        hi = jnp.where(ge, mid, hi)
        lo = jnp.where(ge, lo, mid)
    st_ref[0:1, :] = lo
    st_ref[1:2, :] = jnp.zeros((1, _NBLK), jnp.float32)
    st_ref[2:3, :] = jnp.zeros((1, _NBLK), jnp.float32)

    def scan_body(carry):
        i, _ = carry
        t = st_ref[0:1, :]
        rk2 = st_ref[1:2, :]
        fnd = st_ref[2:3, :]
        nxt = jnp.full((1, _NBLK), big, jnp.float32)
        cnt = jnp.zeros((1, _NBLK), jnp.float32)
        for j in range(_NMC):
            c = d2_ref[pl.ds(j * _MCHUNK, _MCHUNK), :]
            le = c <= t
            nxt = jnp.minimum(
                nxt, jnp.min(jnp.where(le, big, c), axis=0, keepdims=True))
            cnt = cnt + jnp.sum(le.astype(jnp.float32), axis=0, keepdims=True)
        newly = jnp.logical_and(fnd == 0.0, cnt >= kf)
        rk2 = jnp.where(newly, t, rk2)
        fnd = jnp.where(newly, 1.0, fnd)
        st_ref[0:1, :] = jnp.where(fnd > 0.0, t, nxt)
        st_ref[1:2, :] = rk2
        st_ref[2:3, :] = fnd
        return i + 1, jnp.all(fnd > 0.0)

    def scan_cond(carry):
        i, done = carry
        return jnp.logical_and(i < K_NEIGHBORS + 2, jnp.logical_not(done))

    jax.lax.while_loop(scan_cond, scan_body, (jnp.int32(0), False))
    rk2 = jnp.where(st_ref[2:3, :] > 0.0, st_ref[1:2, :], st_ref[0:1, :])
    neg_inv_bw2 = -1.0 / jnp.maximum(rk2, 1e-12)
    acc = jnp.zeros((1, _NBLK), jnp.float32)
    for j in range(_NMC):
        c = d2_ref[pl.ds(j * _MCHUNK, _MCHUNK), :]
        acc = acc + jnp.sum(jnp.exp(c * neg_inv_bw2), axis=0, keepdims=True)
    dens_ref[...] = acc


def _assign_kernel(x_ref, dens_ref, sem_ref):
    dens = dens_ref[...]
    iota = jax.lax.broadcasted_iota(jnp.int32, (1, _N), 1)
    onehots = []
    vals = []
    for _ in range(NUM_CLUSTERS):
        mx = jnp.max(dens)
        idx = jnp.min(jnp.where(dens == mx, iota, jnp.int32(2 ** 30)))
        oh = (iota == idx)
        onehots.append(oh.astype(jnp.float32))
        vals.append(mx)
        dens = jnp.where(oh, jnp.float32(-3.0e38), dens)
    onehot = jnp.concatenate(onehots, axis=0)
    cdens = jnp.stack(vals).reshape(NUM_CLUSTERS, 1)
    centers = jax.lax.dot_general(
        onehot, x_ref[_B - 1], (((1,), (1,)), ((), ())),
        preferred_element_type=jnp.float32, precision=_HI)
    priors = cdens / (jnp.sum(cdens) + 1e-8)
    c2 = jnp.sum(centers * centers, axis=1, keepdims=True)
    rows = []
    for b in range(_B):
        xb = x_ref[b]
        x2 = jnp.sum(xb * xb, axis=0, keepdims=True)
        cm = jnp.dot(centers, xb, preferred_element_type=jnp.float32,
                     precision=_HI)
        d2c = jnp.maximum(c2 + x2 - 2.0 * cm, 1e-12)
        logits = -jnp.sqrt(d2c) / TEMPERATURE
        mxl = jnp.max(logits, axis=0, keepdims=True)
        e = jnp.exp(logits - mxl)
        s = jnp.sum(e, axis=0, keepdims=True)
        rows.append(jnp.sum(priors * e, axis=0, keepdims=True) / s)
    sem_ref[...] = jnp.concatenate(rows, axis=0)


def kernel(features, W_proj, gamma, beta, memory_bank):
    B, C, H, W = features.shape
    f = features.reshape(B, C, H * W)
    g = gamma.reshape(C, 1)
    bt = beta.reshape(C, 1)

    x = pl.pallas_call(
        _project_kernel,
        out_shape=jax.ShapeDtypeStruct((B, C, H * W), jnp.float32),
    )(f, W_proj, g, bt)

    nblocks = (H * W) // _NBLK
    density = pl.pallas_call(
        _density_kernel,
        grid=(nblocks,),
        in_specs=[
            pl.BlockSpec((C, _NBLK), lambda i: (0, i)),
            pl.BlockSpec((MEMORY_SIZE, C), lambda i: (0, 0)),
        ],
        out_specs=pl.BlockSpec((1, _NBLK), lambda i: (0, i)),
        out_shape=jax.ShapeDtypeStruct((1, H * W), jnp.float32),
        scratch_shapes=[pltpu.VMEM((MEMORY_SIZE, _NBLK), jnp.float32),
                        pltpu.VMEM((8, _NBLK), jnp.float32)],
    )(x[B - 1], memory_bank)

    sem = pl.pallas_call(
        _assign_kernel,
        out_shape=jax.ShapeDtypeStruct((B, H * W), jnp.float32),
    )(x, density)

    return sem.reshape(B, 1, H, W)
